# Initial kernel scaffold; baseline (speedup 1.0000x reference)
import jax
import jax.numpy as jnp
from jax.experimental import pallas as pl


def kernel(ranks, suits, rank_table, suit_table):
    raise NotImplementedError("write your pallas kernel here")


# speedup vs baseline: 9.9410x; 9.9410x over previous
import functools

import jax
import jax.numpy as jnp
from jax import lax
from jax.experimental import pallas as pl
from jax.experimental.pallas import tpu as pltpu
from jax.experimental.pallas import tpu_sc as plsc

D = 12


def _build(N, NC, NS, C, LN):
    NW = NC * NS
    per_w = N // NW
    n_chunks = per_w // C
    groups = C // LN

    mesh = plsc.VectorSubcoreMesh(core_axis_name="c", subcore_axis_name="s")

    @functools.partial(
        pl.kernel,
        mesh=mesh,
        compiler_params=pltpu.CompilerParams(needs_layout_passes=False),
        out_type=jax.ShapeDtypeStruct((N * D,), jnp.float32),
        scratch_types=[
            pltpu.VMEM((C,), jnp.int32),
            pltpu.VMEM((C,), jnp.int32),
            pltpu.VMEM((C * D,), jnp.float32),
            pltpu.VMEM((112,), jnp.float32),
            pltpu.VMEM((32,), jnp.float32),
        ],
    )
    def run(ranks_hbm, suits_hbm, rtab_hbm, stab_hbm, out_hbm,
            rb, sb, ob, rtab_v, stab_v):
        wid = lax.axis_index("s") * NC + lax.axis_index("c")
        base = wid * per_w

        pltpu.sync_copy(rtab_hbm, rtab_v)
        pltpu.sync_copy(stab_hbm, stab_v)

        iota12 = lax.iota(jnp.int32, LN) * D

        for g in range(n_chunks):
            off = base + g * C
            pltpu.sync_copy(ranks_hbm.at[pl.ds(off, C)], rb)
            pltpu.sync_copy(suits_hbm.at[pl.ds(off, C)], sb)

            def group(i, _):
                o = i * LN
                r16 = rb[pl.ds(o, LN)]
                s16 = sb[pl.ds(o, LN)]
                r8 = r16 * 8
                s4 = s16 * 4
                pos = o * D + iota12
                for d in range(8):
                    v = plsc.load_gather(rtab_v, [r8 + d])
                    plsc.store_scatter(ob, [pos + d], v)
                for d in range(4):
                    v = plsc.load_gather(stab_v, [s4 + d])
                    plsc.store_scatter(ob, [pos + 8 + d], v)
                return 0

            lax.fori_loop(0, groups, group, 0)
            pltpu.sync_copy(ob, out_hbm.at[pl.ds(off * D, C * D)])

    return run


def kernel(ranks, suits, rank_table, suit_table):
    B, L = ranks.shape
    N = B * L

    info = plsc.get_sparse_core_info()
    NC, NS, LN = info.num_cores, info.num_subcores, info.num_lanes
    C = 3200

    ranks_f = ranks.reshape(N).astype(jnp.int32)
    suits_f = suits.reshape(N).astype(jnp.int32)
    rtab_f = rank_table.reshape(-1).astype(jnp.float32)
    stab_f = jnp.pad(suit_table.reshape(-1).astype(jnp.float32), (0, 12))

    run = _build(N, NC, NS, C, LN)
    out = run(ranks_f, suits_f, rtab_f, stab_f)
    return out.reshape(B, L, D)


# speedup vs baseline: 11.3800x; 1.1447x over previous
import functools

import jax
import jax.numpy as jnp
from jax import lax
from jax.experimental import pallas as pl
from jax.experimental.pallas import tpu as pltpu
from jax.experimental.pallas import tpu_sc as plsc

D = 12


def _build(N, NC, NS, C, LN):
    NW = NC * NS
    per_w = N // NW
    n_chunks = per_w // C
    groups = C // LN

    mesh = plsc.VectorSubcoreMesh(core_axis_name="c", subcore_axis_name="s")

    @functools.partial(
        pl.kernel,
        mesh=mesh,
        compiler_params=pltpu.CompilerParams(needs_layout_passes=False),
        out_type=jax.ShapeDtypeStruct((N * D,), jnp.float32),
        scratch_types=[
            pltpu.VMEM((C,), jnp.int32),
            pltpu.VMEM((C,), jnp.int32),
            pltpu.VMEM((C * D,), jnp.float32),
            pltpu.VMEM((112,), jnp.float32),
            pltpu.VMEM((32,), jnp.float32),
        ],
    )
    def run(ranks_hbm, suits_hbm, rtab_hbm, stab_hbm, out_hbm,
            rb, sb, ob, rtab_v, stab_v):
        wid = lax.axis_index("s") * NC + lax.axis_index("c")
        base = wid * per_w

        pltpu.sync_copy(rtab_hbm, rtab_v)
        pltpu.sync_copy(stab_hbm, stab_v)

        iota12 = lax.iota(jnp.int32, LN) * D

        for g in range(n_chunks):
            off = base + g * C
            pltpu.sync_copy(ranks_hbm.at[pl.ds(off, C)], rb)
            pltpu.sync_copy(suits_hbm.at[pl.ds(off, C)], sb)

            def group(i, _):
                o = i * LN
                r16 = rb[pl.ds(o, LN)]
                s16 = sb[pl.ds(o, LN)]
                r8 = r16 * 8
                s4 = s16 * 4
                pos = o * D + iota12
                vals = [plsc.load_gather(rtab_v, [r8 + d]) for d in range(8)]
                vals += [plsc.load_gather(stab_v, [s4 + d]) for d in range(4)]
                for d in range(D):
                    plsc.store_scatter(ob, [pos + d], vals[d])
                return 0

            lax.fori_loop(0, groups, group, 0)
            pltpu.sync_copy(ob, out_hbm.at[pl.ds(off * D, C * D)])

    return run


def kernel(ranks, suits, rank_table, suit_table):
    B, L = ranks.shape
    N = B * L

    info = plsc.get_sparse_core_info()
    NC, NS, LN = info.num_cores, info.num_subcores, info.num_lanes
    C = 3200

    ranks_f = ranks.reshape(N).astype(jnp.int32)
    suits_f = suits.reshape(N).astype(jnp.int32)
    rtab_f = rank_table.reshape(-1).astype(jnp.float32)
    stab_f = jnp.pad(suit_table.reshape(-1).astype(jnp.float32), (0, 12))

    run = _build(N, NC, NS, C, LN)
    out = run(ranks_f, suits_f, rtab_f, stab_f)
    return out.reshape(B, L, D)


# speedup vs baseline: 57.1228x; 5.0196x over previous
import functools

import jax
import jax.numpy as jnp
from jax import lax
from jax.experimental import pallas as pl
from jax.experimental.pallas import tpu as pltpu
from jax.experimental.pallas import tpu_sc as plsc

D = 12


def _build(B, L, NC, NS, LN):
    NW = NC * NS
    per_w = B // NW
    CB = 128
    n_chunks = per_w // CB
    pairs = (CB // LN) * L

    mesh = plsc.VectorSubcoreMesh(core_axis_name="c", subcore_axis_name="s")

    @functools.partial(
        pl.kernel,
        mesh=mesh,
        compiler_params=pltpu.CompilerParams(needs_layout_passes=False),
        out_type=jax.ShapeDtypeStruct((D, L, B), jnp.float32),
        scratch_types=[
            pltpu.VMEM((CB * L,), jnp.int32),
            pltpu.VMEM((CB * L,), jnp.int32),
            pltpu.VMEM((CB * L,), jnp.int32),
            pltpu.VMEM((CB * L,), jnp.int32),
            pltpu.VMEM((8, 56, CB), jnp.float32),
            pltpu.VMEM((4, 56, CB), jnp.float32),
            pltpu.VMEM((112,), jnp.float32),
            pltpu.VMEM((32,), jnp.float32),
            pltpu.SemaphoreType.DMA,
            pltpu.SemaphoreType.DMA,
            pltpu.SemaphoreType.DMA,
            pltpu.SemaphoreType.DMA,
        ],
    )
    def run(ranks_hbm, suits_hbm, rtab_hbm, stab_hbm, out_hbm,
            rb0, rb1, sb0, sb1, oa, ob, rtab_v, stab_v,
            si0, si1, sa, sb_sem):
        wid = lax.axis_index("s") * NC + lax.axis_index("c")
        base = wid * per_w

        pltpu.sync_copy(rtab_hbm, rtab_v)
        pltpu.sync_copy(stab_hbm, stab_v)

        r_bufs, s_bufs = (rb0, rb1), (sb0, sb1)
        in_sems = (si0, si1)
        iota50 = lax.iota(jnp.int32, LN) * L

        def start_in(g):
            bsel = g % 2
            off = (base + g * CB) * L
            return (
                pltpu.async_copy(ranks_hbm.at[pl.ds(off, CB * L)], r_bufs[bsel], in_sems[bsel]),
                pltpu.async_copy(suits_hbm.at[pl.ds(off, CB * L)], s_bufs[bsel], in_sems[bsel]),
            )

        in_flight = start_in(0)
        a_flight = None
        b_flight = None

        for g in range(n_chunks):
            bsel = g % 2
            for h in in_flight:
                h.wait()
            if g + 1 < n_chunks:
                in_flight = start_in(g + 1)

            rb, sbuf = r_bufs[bsel], s_bufs[bsel]
            b0 = pl.multiple_of((base + g * CB), CB)

            if a_flight is not None:
                a_flight.wait()

            def pair_a(i, _):
                l = i >> 3
                bj0 = (i & 7) * LN
                rv = plsc.load_gather(rb, [iota50 + (bj0 * L + l)])
                r8 = rv * 8
                vals = [plsc.load_gather(rtab_v, [r8 + d]) for d in range(8)]
                for d in range(8):
                    oa[d, l, pl.ds(bj0, LN)] = vals[d]
                return 0

            lax.fori_loop(0, pairs, pair_a, 0)
            a_flight = pltpu.async_copy(
                oa.at[:, pl.ds(0, L), :], out_hbm.at[pl.ds(0, 8), :, pl.ds(b0, CB)], sa
            )

            if b_flight is not None:
                b_flight.wait()

            def pair_b(i, _):
                l = i >> 3
                bj0 = (i & 7) * LN
                sv = plsc.load_gather(sbuf, [iota50 + (bj0 * L + l)])
                s4 = sv * 4
                vals = [plsc.load_gather(stab_v, [s4 + d]) for d in range(4)]
                for d in range(4):
                    ob[d, l, pl.ds(bj0, LN)] = vals[d]
                return 0

            lax.fori_loop(0, pairs, pair_b, 0)
            b_flight = pltpu.async_copy(
                ob.at[:, pl.ds(0, L), :], out_hbm.at[pl.ds(8, 4), :, pl.ds(b0, CB)], sb_sem
            )

        a_flight.wait()
        b_flight.wait()

    return run


def kernel(ranks, suits, rank_table, suit_table):
    B, L = ranks.shape
    N = B * L

    info = plsc.get_sparse_core_info()
    NC, NS, LN = info.num_cores, info.num_subcores, info.num_lanes

    ranks_f = ranks.reshape(N).astype(jnp.int32)
    suits_f = suits.reshape(N).astype(jnp.int32)
    rtab_f = rank_table.reshape(-1).astype(jnp.float32)
    stab_f = jnp.pad(suit_table.reshape(-1).astype(jnp.float32), (0, 12))

    run = _build(B, L, NC, NS, LN)
    out_t = run(ranks_f, suits_f, rtab_f, stab_f)
    return jnp.transpose(out_t, (2, 1, 0))


# speedup vs baseline: 74.2817x; 1.3004x over previous
import functools

import jax
import jax.numpy as jnp
from jax import lax
from jax.experimental import pallas as pl
from jax.experimental.pallas import tpu as pltpu
from jax.experimental.pallas import tpu_sc as plsc

D = 12


def _build(B, L, NC, NS, LN):
    NW = NC * NS
    per_w = B // NW
    CB = 128
    n_chunks = per_w // CB
    pairs = (CB // LN) * L

    mesh = plsc.VectorSubcoreMesh(core_axis_name="c", subcore_axis_name="s")

    @functools.partial(
        pl.kernel,
        mesh=mesh,
        compiler_params=pltpu.CompilerParams(needs_layout_passes=False),
        out_type=jax.ShapeDtypeStruct((D, L, B), jnp.float32),
        scratch_types=[
            pltpu.VMEM((CB * L,), jnp.int32),
            pltpu.VMEM((CB * L,), jnp.int32),
            pltpu.VMEM((CB * L,), jnp.int32),
            pltpu.VMEM((CB * L,), jnp.int32),
            pltpu.VMEM((8, 56, CB), jnp.float32),
            pltpu.VMEM((4, 56, CB), jnp.float32),
            pltpu.VMEM((112,), jnp.float32),
            pltpu.VMEM((32,), jnp.float32),
            pltpu.SemaphoreType.DMA,
            pltpu.SemaphoreType.DMA,
            pltpu.SemaphoreType.DMA,
            pltpu.SemaphoreType.DMA,
        ],
    )
    def run(ranks_hbm, suits_hbm, rtab_hbm, stab_hbm, out_hbm,
            rb0, rb1, sb0, sb1, oa, ob, rtab_v, stab_v,
            si0, si1, sa, sb_sem):
        wid = lax.axis_index("s") * NC + lax.axis_index("c")
        base = wid * per_w

        pltpu.sync_copy(rtab_hbm, rtab_v)
        pltpu.sync_copy(stab_hbm, stab_v)

        r_bufs, s_bufs = (rb0, rb1), (sb0, sb1)
        in_sems = (si0, si1)
        iota50 = lax.iota(jnp.int32, LN) * L

        def start_in(g):
            bsel = g % 2
            off = (base + g * CB) * L
            return (
                pltpu.async_copy(ranks_hbm.at[pl.ds(off, CB * L)], r_bufs[bsel], in_sems[bsel]),
                pltpu.async_copy(suits_hbm.at[pl.ds(off, CB * L)], s_bufs[bsel], in_sems[bsel]),
            )

        in_flight = start_in(0)
        a_flight = None
        b_flight = None

        for g in range(n_chunks):
            bsel = g % 2
            for h in in_flight:
                h.wait()
            if g + 1 < n_chunks:
                in_flight = start_in(g + 1)

            rb, sbuf = r_bufs[bsel], s_bufs[bsel]
            b0 = pl.multiple_of((base + g * CB), CB)

            if a_flight is not None:
                a_flight.wait()

            @plsc.parallel_loop(0, pairs, unroll=2)
            def pair_a(i):
                l = i >> 3
                bj0 = (i & 7) * LN
                rv = plsc.load_gather(rb, [iota50 + (bj0 * L + l)])
                r8 = rv * 8
                vals = [plsc.load_gather(rtab_v, [r8 + d]) for d in range(8)]
                for d in range(8):
                    oa[d, l, pl.ds(bj0, LN)] = vals[d]
            a_flight = pltpu.async_copy(
                oa.at[:, pl.ds(0, L), :], out_hbm.at[pl.ds(0, 8), :, pl.ds(b0, CB)], sa
            )

            if b_flight is not None:
                b_flight.wait()

            @plsc.parallel_loop(0, pairs, unroll=2)
            def pair_b(i):
                l = i >> 3
                bj0 = (i & 7) * LN
                sv = plsc.load_gather(sbuf, [iota50 + (bj0 * L + l)])
                s4 = sv * 4
                vals = [plsc.load_gather(stab_v, [s4 + d]) for d in range(4)]
                for d in range(4):
                    ob[d, l, pl.ds(bj0, LN)] = vals[d]
            b_flight = pltpu.async_copy(
                ob.at[:, pl.ds(0, L), :], out_hbm.at[pl.ds(8, 4), :, pl.ds(b0, CB)], sb_sem
            )

        a_flight.wait()
        b_flight.wait()

    return run


def kernel(ranks, suits, rank_table, suit_table):
    B, L = ranks.shape
    N = B * L

    info = plsc.get_sparse_core_info()
    NC, NS, LN = info.num_cores, info.num_subcores, info.num_lanes

    ranks_f = ranks.reshape(N).astype(jnp.int32)
    suits_f = suits.reshape(N).astype(jnp.int32)
    rtab_f = rank_table.reshape(-1).astype(jnp.float32)
    stab_f = jnp.pad(suit_table.reshape(-1).astype(jnp.float32), (0, 12))

    run = _build(B, L, NC, NS, LN)
    out_t = run(ranks_f, suits_f, rtab_f, stab_f)
    return jnp.transpose(out_t, (2, 1, 0))


# speedup vs baseline: 122.4687x; 1.6487x over previous
import functools

import jax
import jax.numpy as jnp
from jax import lax
from jax.experimental import pallas as pl
from jax.experimental.pallas import tpu as pltpu
from jax.experimental.pallas import tpu_sc as plsc

D = 12


def _build(B, L, NC, NS, LN):
    NW = NC * NS
    per_w = B // NW
    CB = 128
    n_chunks = per_w // CB
    pairs = (CB // LN) * L

    mesh = plsc.VectorSubcoreMesh(core_axis_name="c", subcore_axis_name="s")

    @functools.partial(
        pl.kernel,
        mesh=mesh,
        compiler_params=pltpu.CompilerParams(needs_layout_passes=False),
        out_type=jax.ShapeDtypeStruct((D, L, B), jnp.float32),
        scratch_types=[
            pltpu.VMEM((L, CB), jnp.int32),
            pltpu.VMEM((L, CB), jnp.int32),
            pltpu.VMEM((L, CB), jnp.int32),
            pltpu.VMEM((L, CB), jnp.int32),
            pltpu.VMEM((8, 56, CB), jnp.float32),
            pltpu.VMEM((4, 56, CB), jnp.float32),
            pltpu.VMEM((112,), jnp.float32),
            pltpu.VMEM((32,), jnp.float32),
            pltpu.SemaphoreType.DMA,
            pltpu.SemaphoreType.DMA,
            pltpu.SemaphoreType.DMA,
            pltpu.SemaphoreType.DMA,
            pltpu.SemaphoreType.DMA,
            pltpu.SemaphoreType.DMA,
        ],
    )
    def run(ranks_hbm, suits_hbm, rtab_hbm, stab_hbm, out_hbm,
            rb0, rb1, sb0, sb1, oa, ob, rtab_v, stab_v,
            si0, si1, sj0, sj1, sa, sbs):
        wid = lax.axis_index("s") * NC + lax.axis_index("c")
        base = wid * per_w

        pltpu.sync_copy(rtab_hbm, rtab_v)
        pltpu.sync_copy(stab_hbm, stab_v)

        r_bufs, s_bufs = (rb0, rb1), (sb0, sb1)
        r_sems, s_sems = (si0, si1), (sj0, sj1)
        iota16 = lax.iota(jnp.int32, LN)

        def start_r(g):
            sel = g % 2
            return pltpu.async_copy(
                ranks_hbm.at[:, pl.ds(base + g * CB, CB)], r_bufs[sel], r_sems[sel])

        def start_s(g):
            sel = g % 2
            return pltpu.async_copy(
                suits_hbm.at[:, pl.ds(base + g * CB, CB)], s_bufs[sel], s_sems[sel])

        r_fl, s_fl = start_r(0), start_s(0)
        a_fl = b_fl = None

        for g in range(n_chunks):
            sel = g % 2
            rb_c, sb_c = r_bufs[sel], s_bufs[sel]
            b0 = pl.multiple_of(base + g * CB, CB)

            r_fl.wait()
            if g + 1 < n_chunks:
                r_fl = start_r(g + 1)
            if a_fl is not None:
                a_fl.wait()

            @plsc.parallel_loop(0, pairs, unroll=2)
            def pair_a(i):
                l = i >> 3
                bj0 = (i & 7) * LN
                rv = plsc.load_gather(
                    rb_c, [jnp.full((LN,), l, jnp.int32), iota16 + bj0])
                r8 = rv * 8
                vals = [plsc.load_gather(rtab_v, [r8 + d]) for d in range(8)]
                for d in range(8):
                    oa[d, l, pl.ds(bj0, LN)] = vals[d]

            a_fl = pltpu.async_copy(
                oa.at[:, pl.ds(0, L), :], out_hbm.at[pl.ds(0, 8), :, pl.ds(b0, CB)], sa)

            s_fl.wait()
            if g + 1 < n_chunks:
                s_fl = start_s(g + 1)
            if b_fl is not None:
                b_fl.wait()

            @plsc.parallel_loop(0, pairs, unroll=2)
            def pair_b(i):
                l = i >> 3
                bj0 = (i & 7) * LN
                sv = plsc.load_gather(
                    sb_c, [jnp.full((LN,), l, jnp.int32), iota16 + bj0])
                s4 = sv * 4
                vals = [plsc.load_gather(stab_v, [s4 + d]) for d in range(4)]
                for d in range(4):
                    ob[d, l, pl.ds(bj0, LN)] = vals[d]

            b_fl = pltpu.async_copy(
                ob.at[:, pl.ds(0, L), :], out_hbm.at[pl.ds(8, 4), :, pl.ds(b0, CB)], sbs)

        a_fl.wait()
        b_fl.wait()

    return run


def kernel(ranks, suits, rank_table, suit_table):
    B, L = ranks.shape

    info = plsc.get_sparse_core_info()
    NC, NS, LN = info.num_cores, info.num_subcores, info.num_lanes

    ranks_t = jnp.transpose(ranks.astype(jnp.int32))
    suits_t = jnp.transpose(suits.astype(jnp.int32))
    rtab_f = rank_table.reshape(-1).astype(jnp.float32)
    stab_f = jnp.pad(suit_table.reshape(-1).astype(jnp.float32), (0, 12))

    run = _build(B, L, NC, NS, LN)
    out_t = run(ranks_t, suits_t, rtab_f, stab_f)
    return jnp.transpose(out_t, (2, 1, 0))
